# Initial kernel scaffold; baseline (speedup 1.0000x reference)
import jax
import jax.numpy as jnp
from jax.experimental import pallas as pl


def kernel(x, edge_index, edge_attr, batch, node_emb, edge_emb, W1, b1, W2, b2, eps, gamma, beta):
    raise NotImplementedError("write your pallas kernel here")


# speedup vs baseline: 1.5265x; 1.5265x over previous
import functools

import jax
import jax.numpy as jnp
from jax import lax
from jax.experimental import pallas as pl
from jax.experimental.pallas import tpu as pltpu
from jax.experimental.pallas import tpu_sc as plsc

N = 10000
E = 160000
H = 256
HH = 128
L = 4
NGRAPH = 64
NSC = 2
NTEC = 16
CHUNK = 128

NP = 10240
EPT = 79 * CHUNK
EP = NTEC * EPT
TRASH = N + 16

_mesh = lambda: plsc.VectorSubcoreMesh(
    core_axis_name="c", subcore_axis_name="s", num_cores=NSC, num_subcores=NTEC)


def _rows_relu_add(rows, other, r):
    for j in range(HH // 16):
        sl = pl.ds(j * 16, 16)
        rows[r, sl] = jnp.maximum(rows[r, sl] + other[r, sl], 0.0)


def _rows_add(acc, rows, r):
    for j in range(HH // 16):
        sl = pl.ds(j * 16, 16)
        acc[r, sl] = acc[r, sl] + rows[r, sl]


def _embed_body(x_cols, attr_cols, ne_a, ne_b, ee_a, ee_b,
                h_a, h_b, e_a, e_b, idx, rows, acc, sem):
    c = lax.axis_index("c")
    s = lax.axis_index("s")

    def gsum(K, cols_ref, stride, table_ref, out_ref, nchunks, base):
        def chunk_body(i, carry):
            off = base + i * CHUNK
            for k in range(K):
                pltpu.sync_copy(cols_ref.at[pl.ds(k * stride + off, CHUNK)], idx)
                if k == 0:
                    pltpu.async_copy(table_ref.at[idx], acc, sem).wait()
                else:
                    pltpu.async_copy(table_ref.at[idx], rows, sem).wait()
                    lax.fori_loop(0, CHUNK,
                                  lambda r, cc: (_rows_add(acc, rows, r), cc)[1],
                                  0, unroll=2)
            pltpu.sync_copy(acc, out_ref.at[pl.ds(off, CHUNK), :])
            return carry
        lax.fori_loop(0, nchunks, chunk_body, 0)

    def run(ne, ee, h_out, e_out):
        gsum(9, x_cols, NP, ne, h_out, NP // NTEC // CHUNK, s * (NP // NTEC))
        gsum(3, attr_cols, EP, ee, e_out, EPT // CHUNK, s * EPT)

    @pl.when(c == 0)
    def _():
        run(ne_a, ee_a, h_a, e_a)

    @pl.when(c == 1)
    def _():
        run(ne_b, ee_b, h_b, e_b)


def _embed_call(x_cols, attr_cols, ne_a, ne_b, ee_a, ee_b):
    f = pl.kernel(
        _embed_body,
        out_type=[jax.ShapeDtypeStruct((NP, HH), jnp.float32),
                  jax.ShapeDtypeStruct((NP, HH), jnp.float32),
                  jax.ShapeDtypeStruct((EP, HH), jnp.float32),
                  jax.ShapeDtypeStruct((EP, HH), jnp.float32)],
        mesh=_mesh(),
        scratch_types=[pltpu.VMEM((CHUNK,), jnp.int32),
                       pltpu.VMEM((CHUNK, HH), jnp.float32),
                       pltpu.VMEM((CHUNK, HH), jnp.float32),
                       pltpu.SemaphoreType.DMA],
    )
    return f(x_cols, attr_cols, ne_a, ne_b, ee_a, ee_b)


def _msg_body(h_a, h_b, e_a, e_b, src, dst, zeros,
              agg_a, agg_b, idx_s, idx_d, rows, erow, agg_sh, sem):
    c = lax.axis_index("c")
    s = lax.axis_index("s")
    stripe = NP // NTEC

    def run(h_h, e_h, agg_h):
        pltpu.sync_copy(zeros.at[pl.ds(s * stripe, stripe), :],
                        agg_sh.at[pl.ds(s * stripe, stripe), :])
        plsc.subcore_barrier()

        base = s * EPT

        def chunk_body(i, carry):
            off = base + i * CHUNK
            pltpu.sync_copy(src.at[pl.ds(off, CHUNK)], idx_s)
            pltpu.async_copy(h_h.at[idx_s], rows, sem).wait()
            pltpu.sync_copy(e_h.at[pl.ds(off, CHUNK), :], erow)
            lax.fori_loop(0, CHUNK,
                          lambda r, cc: (_rows_relu_add(rows, erow, r), cc)[1],
                          0, unroll=2)
            pltpu.sync_copy(dst.at[pl.ds(off, CHUNK)], idx_d)
            pltpu.sync_copy(rows, agg_sh.at[idx_d], add=True)
            return carry

        lax.fori_loop(0, EPT // CHUNK, chunk_body, 0)
        plsc.subcore_barrier()
        pltpu.sync_copy(agg_sh.at[pl.ds(s * stripe, stripe), :],
                        agg_h.at[pl.ds(s * stripe, stripe), :])

    @pl.when(c == 0)
    def _():
        run(h_a, e_a, agg_a)

    @pl.when(c == 1)
    def _():
        run(h_b, e_b, agg_b)


def _msg_call(h_a, h_b, e_a, e_b, src, dst, zeros):
    f = pl.kernel(
        _msg_body,
        out_type=[jax.ShapeDtypeStruct((NP, HH), jnp.float32),
                  jax.ShapeDtypeStruct((NP, HH), jnp.float32)],
        mesh=_mesh(),
        scratch_types=[pltpu.VMEM((CHUNK,), jnp.int32),
                       pltpu.VMEM((CHUNK,), jnp.int32),
                       pltpu.VMEM((CHUNK, HH), jnp.float32),
                       pltpu.VMEM((CHUNK, HH), jnp.float32),
                       pltpu.VMEM_SHARED((NP, HH), jnp.float32),
                       pltpu.SemaphoreType.DMA],
    )
    return f(h_a, h_b, e_a, e_b, src, dst, zeros)


BN = 400
NT = N // BN

_dot = functools.partial(jax.lax.dot_general,
                         precision=jax.lax.Precision.HIGHEST,
                         preferred_element_type=jnp.float32)


def _mm_kernel(h_a, h_b, g_a, g_b, w1, w2, par, h2_ref, st_ref):
    i = pl.program_id(0)
    h = jnp.concatenate([h_a[...], h_b[...]], axis=1)
    agg = jnp.concatenate([g_a[...], g_b[...]], axis=1)
    p = par[...]
    pre = h * p[0] + agg
    z = jnp.maximum(_dot(pre, w1[...], (((1,), (0,)), ((), ()))) + p[1], 0.0)
    h2 = _dot(z, w2[...], (((1,), (0,)), ((), ()))) + p[2]
    h2_ref[...] = h2
    s1 = jnp.sum(h2, axis=0)
    s2 = jnp.sum(h2 * h2, axis=0)
    upd = jnp.concatenate([s1[None], s2[None], jnp.zeros((6, H), jnp.float32)], 0)

    @pl.when(i == 0)
    def _():
        st_ref[...] = jnp.zeros((8, H), jnp.float32)

    st_ref[...] += upd


def _mm_call(h_a, h_b, g_a, g_b, w1, w2, par):
    half = lambda: pl.BlockSpec((BN, HH), lambda i: (i, 0))
    return pl.pallas_call(
        _mm_kernel,
        grid=(NT,),
        in_specs=[half(), half(), half(), half(),
                  pl.BlockSpec((H, H), lambda i: (0, 0)),
                  pl.BlockSpec((H, H), lambda i: (0, 0)),
                  pl.BlockSpec((8, H), lambda i: (0, 0))],
        out_specs=[pl.BlockSpec((BN, H), lambda i: (i, 0)),
                   pl.BlockSpec((8, H), lambda i: (0, 0))],
        out_shape=[jax.ShapeDtypeStruct((N, H), jnp.float32),
                   jax.ShapeDtypeStruct((8, H), jnp.float32)],
    )(h_a, h_b, g_a, g_b, w1, w2, par)


def _apply_kernel(h2, h_a, h_b, par, oa_ref, ob_ref):
    p = par[...]
    hn = jnp.maximum(h2[...] * p[0] + p[1], 0.0)
    hn = hn + jnp.concatenate([h_a[...], h_b[...]], axis=1)
    oa_ref[...] = hn[:, :HH]
    ob_ref[...] = hn[:, HH:]


def _apply_call(h2, h_a, h_b, par):
    half = lambda: pl.BlockSpec((BN, HH), lambda i: (i, 0))
    return pl.pallas_call(
        _apply_kernel,
        grid=(NT,),
        in_specs=[pl.BlockSpec((BN, H), lambda i: (i, 0)), half(), half(),
                  pl.BlockSpec((8, H), lambda i: (0, 0))],
        out_specs=[half(), half()],
        out_shape=[jax.ShapeDtypeStruct((NP, HH), jnp.float32),
                   jax.ShapeDtypeStruct((NP, HH), jnp.float32)],
    )(h2, h_a, h_b, par)


def _final_kernel(h2, h_a, h_b, par, bt, hf_ref, g_ref):
    i = pl.program_id(0)
    p = par[...]
    hn = jnp.maximum(h2[...] * p[0] + p[1], 0.0)
    hn = hn + jnp.concatenate([h_a[...], h_b[...]], axis=1)
    hf_ref[...] = hn
    gids = bt[...]
    onehot = (gids == lax.broadcasted_iota(jnp.int32, (BN, NGRAPH), 1)
              ).astype(jnp.float32)
    gt = _dot(onehot, hn, (((0,), (0,)), ((), ())))

    @pl.when(i == 0)
    def _():
        g_ref[...] = jnp.zeros((NGRAPH, H), jnp.float32)

    g_ref[...] += gt


def _final_call(h2, h_a, h_b, par, batch2d):
    half = lambda: pl.BlockSpec((BN, HH), lambda i: (i, 0))
    return pl.pallas_call(
        _final_kernel,
        grid=(NT,),
        in_specs=[pl.BlockSpec((BN, H), lambda i: (i, 0)), half(), half(),
                  pl.BlockSpec((8, H), lambda i: (0, 0)),
                  pl.BlockSpec((BN, 1), lambda i: (i, 0))],
        out_specs=[pl.BlockSpec((BN, H), lambda i: (i, 0)),
                   pl.BlockSpec((NGRAPH, H), lambda i: (0, 0))],
        out_shape=[jax.ShapeDtypeStruct((N, H), jnp.float32),
                   jax.ShapeDtypeStruct((NGRAPH, H), jnp.float32)],
    )(h2, h_a, h_b, par, batch2d)


def kernel(x, edge_index, edge_attr, batch, node_emb, edge_emb,
           W1, b1, W2, b2, eps, gamma, beta):
    f32 = jnp.float32
    x_cols = jnp.pad(x.astype(jnp.int32).T, ((0, 0), (0, NP - N))).reshape(-1)
    attr_cols = jnp.pad(edge_attr.astype(jnp.int32).T,
                        ((0, 0), (0, EP - E))).reshape(-1)
    src = jnp.pad(edge_index[0].astype(jnp.int32), (0, EP - E))
    dst = jnp.pad(edge_index[1].astype(jnp.int32), (0, EP - E),
                  constant_values=TRASH)
    ne_a, ne_b = node_emb[:, :HH], node_emb[:, HH:]
    ee_a, ee_b = edge_emb[:, :HH], edge_emb[:, HH:]
    zeros = jnp.zeros((NP, HH), f32)
    batch2d = batch.astype(jnp.int32).reshape(N, 1)

    h_a, h_b, e_a, e_b = _embed_call(x_cols, attr_cols, ne_a, ne_b, ee_a, ee_b)

    ones_h = jnp.ones((H,), f32)
    h_full = None
    g = None
    for l in range(L):
        agg_a, agg_b = _msg_call(h_a, h_b, e_a, e_b, src, dst, zeros)
        par_mm = jnp.concatenate(
            [((1.0 + eps[l]) * ones_h)[None], b1[l][None], b2[l][None],
             jnp.zeros((5, H), f32)], 0)
        h2, st = _mm_call(h_a, h_b, agg_a, agg_b, W1[l], W2[l], par_mm)
        mean = st[0] / N
        var = st[1] / N - mean * mean
        scale = gamma[l] / jnp.sqrt(var + 1e-5)
        shift = beta[l] - mean * scale
        par_ap = jnp.concatenate(
            [scale[None], shift[None], jnp.zeros((6, H), f32)], 0)
        if l < L - 1:
            h_a, h_b = _apply_call(h2, h_a, h_b, par_ap)
        else:
            h_full, g = _final_call(h2, h_a, h_b, par_ap, batch2d)
    return (h_full, g, batch)
